# baseline (speedup 1.0000x reference)
import functools

import jax
import jax.numpy as jnp
import numpy as np
from jax import lax
from jax.experimental import pallas as pl
from jax.experimental.pallas import tpu as pltpu
from jax.experimental.pallas import tpu_sc as plsc

GRID = 32
V = GRID ** 3
N_PTS = 262144
B, C = 2, 64
R = B * C
NC, NS = 2, 16
NW = NC * NS
ROWS_PER_W = R // NW
L = 16

K = 2048
U = 8
VS_EPS = np.float32(np.float32(0.1) + np.float32(1e-6))

_TRIPLE_SUM = np.zeros((384, 128), np.float32)
_TRIPLE_SUM[np.arange(384), np.arange(384) // 3] = 1.0
_COEFF = np.tile(np.array([GRID * GRID, GRID, 1], np.float32), 128)


def _seg_body(pts_ref, coeff_ref, tmat_ref, seg_ref):
    coord = (pts_ref[...] / VS_EPS).astype(jnp.int32).astype(jnp.float32)
    scaled = coord * coeff_ref[...]
    seg = jax.lax.dot_general(scaled, tmat_ref[...], (((1,), (0,)), ((), ())),
                              preferred_element_type=jnp.float32)
    seg_ref[...] = seg.astype(jnp.int32)


@jax.jit
def _segments(pts_flat):
    return pl.pallas_call(
        _seg_body,
        out_shape=jax.ShapeDtypeStruct((N_PTS // 128, 128), jnp.int32),
    )(pts_flat.reshape(N_PTS // 128, 384), jnp.asarray(_COEFF[None, :]),
      jnp.asarray(_TRIPLE_SUM))


def _body(seg_hbm, feat_hbm, out_hbm, seg_sh, acc, seg_buf, fbuf_a, fbuf_b,
          sseg0, sseg1, sfa0, sfa1, sfb0, sfb1):
    c = lax.axis_index("c")
    s = lax.axis_index("s")
    wid = c * NS + s

    ones = jnp.ones((L,), jnp.float32)
    zeros = jnp.zeros((L,), jnp.float32)

    seg_per_sub = N_PTS // NS
    pltpu.sync_copy(seg_hbm.at[pl.ds(s * seg_per_sub, seg_per_sub)],
                    seg_sh.at[pl.ds(s * seg_per_sub, seg_per_sub)])
    plsc.subcore_barrier()

    nch = N_PTS // K
    sems = ((sseg0, sfa0, sfb0), (sseg1, sfa1, sfb1))

    for p in range(2):
        row0 = wid * ROWS_PER_W + 2 * p
        nzero = 3 * V if p == 0 else 2 * V

        def zbody(i, _):
            for u in range(U):
                acc[pl.ds((i * U + u) * L, L)] = zeros
            return 0

        lax.fori_loop(0, nzero // (L * U), zbody, 0)

        def issue(j, b):
            off = j * K
            ss, sa, sb = sems[b]
            pltpu.async_copy(seg_sh.at[pl.ds(off, K)], seg_buf.at[b], ss)
            pltpu.async_copy(feat_hbm.at[pl.ds(row0 * N_PTS + off, K)],
                             fbuf_a.at[b], sa)
            pltpu.async_copy(feat_hbm.at[pl.ds((row0 + 1) * N_PTS + off, K)],
                             fbuf_b.at[b], sb)

        def wait(b):
            ss, sa, sb = sems[b]
            pltpu.make_async_copy(seg_sh.at[pl.ds(0, K)], seg_buf.at[b],
                                  ss).wait()
            pltpu.make_async_copy(feat_hbm.at[pl.ds(0, K)], fbuf_a.at[b],
                                  sa).wait()
            pltpu.make_async_copy(feat_hbm.at[pl.ds(0, K)], fbuf_b.at[b],
                                  sb).wait()

        def compute(b, with_counts):
            def ibody(i, _):
                for u in range(U):
                    bb = (i * U + u) * L
                    seg = seg_buf[b, pl.ds(bb, L)]
                    va = fbuf_a[b, pl.ds(bb, L)]
                    vb = fbuf_b[b, pl.ds(bb, L)]
                    plsc.addupdate_scatter(acc, [seg], va)
                    plsc.addupdate_scatter(acc, [seg + V], vb)
                    if with_counts:
                        plsc.addupdate_scatter(acc, [seg + 2 * V], ones)
                return 0

            lax.fori_loop(0, K // (L * U), ibody, 0)

        issue(0, 0)

        def mchunk(jj, _):
            for b in range(2):
                j = jj * 2 + b
                nxt = j + 1
                nxt = jnp.where(nxt >= nch, 0, nxt)
                issue(nxt, b ^ 1)
                wait(b)
                compute(b, p == 0)
            return 0

        lax.fori_loop(0, nch // 2, mchunk, 0)
        wait(0)

        def fbody(i, _):
            for u in range(4):
                bb = (i * 4 + u) * L
                cnt = jnp.maximum(acc[pl.ds(2 * V + bb, L)], 1.0)
                acc[pl.ds(bb, L)] = acc[pl.ds(bb, L)] / cnt
                acc[pl.ds(V + bb, L)] = acc[pl.ds(V + bb, L)] / cnt
            return 0

        lax.fori_loop(0, V // (L * 4), fbody, 0)
        pltpu.sync_copy(acc.at[pl.ds(0, V)], out_hbm.at[pl.ds(row0 * V, V)])
        pltpu.sync_copy(acc.at[pl.ds(V, V)],
                        out_hbm.at[pl.ds((row0 + 1) * V, V)])


@jax.jit
def _voxelize(seg, feat_flat):
    mesh = plsc.VectorSubcoreMesh(core_axis_name="c", subcore_axis_name="s")
    return pl.kernel(
        _body,
        out_type=jax.ShapeDtypeStruct((R * V,), jnp.float32),
        mesh=mesh,
        compiler_params=pltpu.CompilerParams(needs_layout_passes=False),
        scratch_types=[
            pltpu.VMEM_SHARED((N_PTS,), jnp.int32),
            pltpu.VMEM((3 * V,), jnp.float32),
            pltpu.VMEM((2, K), jnp.int32),
            pltpu.VMEM((2, K), jnp.float32),
            pltpu.VMEM((2, K), jnp.float32),
            pltpu.SemaphoreType.DMA,
            pltpu.SemaphoreType.DMA,
            pltpu.SemaphoreType.DMA,
            pltpu.SemaphoreType.DMA,
            pltpu.SemaphoreType.DMA,
            pltpu.SemaphoreType.DMA,
        ],
    )(seg, feat_flat)


def kernel(points, features):
    seg = _segments(points.reshape(-1)).reshape(-1)
    out = _voxelize(seg, features.reshape(-1))
    return out.reshape(B, C, GRID, GRID, GRID)


# speedup vs baseline: 1.1684x; 1.1684x over previous
import functools

import jax
import jax.numpy as jnp
import numpy as np
from jax import lax
from jax.experimental import pallas as pl
from jax.experimental.pallas import tpu as pltpu
from jax.experimental.pallas import tpu_sc as plsc

GRID = 32
V = GRID ** 3
N_PTS = 262144
B, C = 2, 64
R = B * C
NC, NS = 2, 16
NW = NC * NS
ROWS_PER_W = R // NW
L = 16

K = 2048
U = 8
VS_EPS = np.float32(np.float32(0.1) + np.float32(1e-6))

_TRIPLE_SUM = np.zeros((384, 128), np.float32)
_TRIPLE_SUM[np.arange(384), np.arange(384) // 3] = 1.0
_COEFF = np.tile(np.array([GRID * GRID, GRID, 1], np.float32), 128)


def _seg_body(pts_ref, coeff_ref, tmat_ref, seg_ref):
    coord = (pts_ref[...] / VS_EPS).astype(jnp.int32).astype(jnp.float32)
    scaled = coord * coeff_ref[...]
    seg = jax.lax.dot_general(scaled, tmat_ref[...], (((1,), (0,)), ((), ())),
                              preferred_element_type=jnp.float32)
    seg_ref[...] = seg.astype(jnp.int32)


@jax.jit
def _segments(pts_flat):
    return pl.pallas_call(
        _seg_body,
        out_shape=jax.ShapeDtypeStruct((N_PTS // 128, 128), jnp.int32),
    )(pts_flat.reshape(N_PTS // 128, 384), jnp.asarray(_COEFF[None, :]),
      jnp.asarray(_TRIPLE_SUM))


def _body(seg_hbm, feat_hbm, out_hbm, seg_sh, acc, seg_buf, fbuf,
          sseg0, sseg1, sfa0, sfa1, sfb0, sfb1):
    c = lax.axis_index("c")
    s = lax.axis_index("s")
    wid = c * NS + s
    tile = wid // 2
    half = wid % 2

    ones = jnp.ones((L,), jnp.float32)
    zeros = jnp.zeros((L,), jnp.float32)

    seg_per_sub = N_PTS // NS
    pltpu.sync_copy(seg_hbm.at[pl.ds(s * seg_per_sub, seg_per_sub)],
                    seg_sh.at[pl.ds(s * seg_per_sub, seg_per_sub)])
    plsc.subcore_barrier()

    nch = N_PTS // K
    sems = ((sseg0, sfa0, sfb0), (sseg1, sfa1, sfb1))

    for p in range(2):
        row0 = 8 * tile + 4 * half + 2 * p
        nzero = 3 * V if p == 0 else 2 * V

        def zbody(i, _):
            for u in range(U):
                acc[pl.ds((i * U + u) * L, L)] = zeros
            return 0

        lax.fori_loop(0, nzero // (L * U), zbody, 0)

        def issue(j, b):
            ss, sa, sb = sems[b]
            pltpu.async_copy(seg_sh.at[pl.ds(j * K, K)], seg_buf.at[b], ss)
            pltpu.async_copy(
                feat_hbm.at[pl.ds(tile * 2048 + j * (K // 128), K // 128),
                            pl.ds(4 * half + 2 * p, 2), :],
                fbuf.at[b], sa)

        def wait(b):
            ss, sa, sb = sems[b]
            pltpu.make_async_copy(seg_sh.at[pl.ds(0, K)], seg_buf.at[b],
                                  ss).wait()
            pltpu.make_async_copy(feat_hbm.at[pl.ds(0, K // 128),
                                              pl.ds(0, 2), :],
                                  fbuf.at[b], sa).wait()

        def compute(b, with_counts):
            def ibody(i, _):
                for u in range(U):
                    seg = seg_buf[b, pl.ds(i * 128 + u * L, L)]
                    va = fbuf[b, i, 0, pl.ds(u * L, L)]
                    vb = fbuf[b, i, 1, pl.ds(u * L, L)]
                    plsc.addupdate_scatter(acc, [seg], va)
                    plsc.addupdate_scatter(acc, [seg + V], vb)
                    if with_counts:
                        plsc.addupdate_scatter(acc, [seg + 2 * V], ones)
                return 0

            lax.fori_loop(0, K // 128, ibody, 0)

        issue(0, 0)

        def mchunk(jj, _):
            for b in range(2):
                j = jj * 2 + b
                nxt = j + 1
                nxt = jnp.where(nxt >= nch, 0, nxt)
                issue(nxt, b ^ 1)
                wait(b)
                compute(b, p == 0)
            return 0

        lax.fori_loop(0, nch // 2, mchunk, 0)
        wait(0)

        def fbody(i, _):
            for u in range(4):
                bb = (i * 4 + u) * L
                cnt = jnp.maximum(acc[pl.ds(2 * V + bb, L)], 1.0)
                acc[pl.ds(bb, L)] = acc[pl.ds(bb, L)] / cnt
                acc[pl.ds(V + bb, L)] = acc[pl.ds(V + bb, L)] / cnt
            return 0

        lax.fori_loop(0, V // (L * 4), fbody, 0)
        pltpu.sync_copy(acc.at[pl.ds(0, V)], out_hbm.at[pl.ds(row0 * V, V)])
        pltpu.sync_copy(acc.at[pl.ds(V, V)],
                        out_hbm.at[pl.ds((row0 + 1) * V, V)])


@jax.jit
def _voxelize(seg, feat_flat):
    mesh = plsc.VectorSubcoreMesh(core_axis_name="c", subcore_axis_name="s")
    return pl.kernel(
        _body,
        out_type=jax.ShapeDtypeStruct((R * V,), jnp.float32),
        mesh=mesh,
        compiler_params=pltpu.CompilerParams(needs_layout_passes=False),
        scratch_types=[
            pltpu.VMEM_SHARED((N_PTS,), jnp.int32),
            pltpu.VMEM((3 * V,), jnp.float32),
            pltpu.VMEM((2, K), jnp.int32),
            pltpu.VMEM((2, K // 128, 2, 128), jnp.float32),
            pltpu.SemaphoreType.DMA,
            pltpu.SemaphoreType.DMA,
            pltpu.SemaphoreType.DMA,
            pltpu.SemaphoreType.DMA,
            pltpu.SemaphoreType.DMA,
            pltpu.SemaphoreType.DMA,
        ],
    )(seg, feat_flat)


def kernel(points, features):
    seg = _segments(points.reshape(-1)).reshape(-1)
    feat_view = (features.reshape(B, C // 8, 8, N_PTS // 128, 128)
                 .transpose(0, 1, 3, 2, 4)
                 .reshape(B * (C // 8) * (N_PTS // 128), 8, 128))
    out = _voxelize(seg, feat_view)
    return out.reshape(B, C, GRID, GRID, GRID)


# speedup vs baseline: 1.1697x; 1.0010x over previous
import functools

import jax
import jax.numpy as jnp
import numpy as np
from jax import lax
from jax.experimental import pallas as pl
from jax.experimental.pallas import tpu as pltpu
from jax.experimental.pallas import tpu_sc as plsc

GRID = 32
V = GRID ** 3
N_PTS = 262144
B, C = 2, 64
R = B * C
NC, NS = 2, 16
NW = NC * NS
ROWS_PER_W = R // NW
L = 16

K = 2048
U = 8
VS_EPS = np.float32(np.float32(0.1) + np.float32(1e-6))

_SEG_BLK = 4096


def _seg_body(pts_ref, seg_ref):
    xi = (pts_ref[:, 0] / VS_EPS).astype(jnp.int32)
    yi = (pts_ref[:, 1] / VS_EPS).astype(jnp.int32)
    zi = (pts_ref[:, 2] / VS_EPS).astype(jnp.int32)
    seg = xi * (GRID * GRID) + yi * GRID + zi
    seg_ref[...] = seg.reshape(_SEG_BLK // 128, 128)


@jax.jit
def _segments(points):
    return pl.pallas_call(
        _seg_body,
        grid=(N_PTS // _SEG_BLK,),
        in_specs=[pl.BlockSpec((_SEG_BLK, 3), lambda i: (i, 0))],
        out_specs=pl.BlockSpec((_SEG_BLK // 128, 128), lambda i: (i, 0)),
        out_shape=jax.ShapeDtypeStruct((N_PTS // 128, 128), jnp.int32),
    )(points)


def _tr_body(acc_ref, out_ref):
    out_ref[0, :, :] = acc_ref[...].T


@jax.jit
def _to_cminor(flat):
    return pl.pallas_call(
        _tr_body,
        grid=(B, 8),
        in_specs=[pl.BlockSpec((C, V // 8), lambda b, j: (b, j))],
        out_specs=pl.BlockSpec((1, V // 8, C), lambda b, j: (b, j, 0)),
        out_shape=jax.ShapeDtypeStruct((B, V, C), jnp.float32),
    )(flat.reshape(R, V))


def _body(seg_hbm, feat_hbm, out_hbm, seg_sh, acc, seg_buf, fbuf,
          sseg0, sseg1, sfa0, sfa1, sfb0, sfb1):
    c = lax.axis_index("c")
    s = lax.axis_index("s")
    wid = c * NS + s
    tile = wid // 2
    half = wid % 2

    ones = jnp.ones((L,), jnp.float32)
    zeros = jnp.zeros((L,), jnp.float32)

    seg_per_sub = N_PTS // NS
    pltpu.sync_copy(seg_hbm.at[pl.ds(s * seg_per_sub, seg_per_sub)],
                    seg_sh.at[pl.ds(s * seg_per_sub, seg_per_sub)])
    plsc.subcore_barrier()

    nch = N_PTS // K
    sems = ((sseg0, sfa0, sfb0), (sseg1, sfa1, sfb1))

    for p in range(2):
        row0 = 8 * tile + 4 * half + 2 * p
        nzero = 3 * V if p == 0 else 2 * V

        def zbody(i, _):
            for u in range(U):
                acc[pl.ds((i * U + u) * L, L)] = zeros
            return 0

        lax.fori_loop(0, nzero // (L * U), zbody, 0)

        def issue(j, b):
            ss, sa, sb = sems[b]
            pltpu.async_copy(seg_sh.at[pl.ds(j * K, K)], seg_buf.at[b], ss)
            pltpu.async_copy(
                feat_hbm.at[pl.ds(tile * 2048 + j * (K // 128), K // 128),
                            pl.ds(4 * half + 2 * p, 2), :],
                fbuf.at[b], sa)

        def wait(b):
            ss, sa, sb = sems[b]
            pltpu.make_async_copy(seg_sh.at[pl.ds(0, K)], seg_buf.at[b],
                                  ss).wait()
            pltpu.make_async_copy(feat_hbm.at[pl.ds(0, K // 128),
                                              pl.ds(0, 2), :],
                                  fbuf.at[b], sa).wait()

        def compute(b, with_counts):
            def ibody(i, _):
                for u in range(U):
                    seg = seg_buf[b, pl.ds(i * 128 + u * L, L)]
                    va = fbuf[b, i, 0, pl.ds(u * L, L)]
                    vb = fbuf[b, i, 1, pl.ds(u * L, L)]
                    plsc.addupdate_scatter(acc, [seg], va)
                    plsc.addupdate_scatter(acc, [seg + V], vb)
                    if with_counts:
                        plsc.addupdate_scatter(acc, [seg + 2 * V], ones)
                return 0

            lax.fori_loop(0, K // 128, ibody, 0)

        issue(0, 0)

        def mchunk(jj, _):
            for b in range(2):
                j = jj * 2 + b
                nxt = j + 1
                nxt = jnp.where(nxt >= nch, 0, nxt)
                issue(nxt, b ^ 1)
                wait(b)
                compute(b, p == 0)
            return 0

        lax.fori_loop(0, nch // 2, mchunk, 0)
        wait(0)

        def fbody(i, _):
            for u in range(4):
                bb = (i * 4 + u) * L
                cnt = jnp.maximum(acc[pl.ds(2 * V + bb, L)], 1.0)
                acc[pl.ds(bb, L)] = acc[pl.ds(bb, L)] / cnt
                acc[pl.ds(V + bb, L)] = acc[pl.ds(V + bb, L)] / cnt
            return 0

        lax.fori_loop(0, V // (L * 4), fbody, 0)
        pltpu.sync_copy(acc.at[pl.ds(0, V)], out_hbm.at[pl.ds(row0 * V, V)])
        pltpu.sync_copy(acc.at[pl.ds(V, V)],
                        out_hbm.at[pl.ds((row0 + 1) * V, V)])


@jax.jit
def _voxelize(seg, feat_flat):
    mesh = plsc.VectorSubcoreMesh(core_axis_name="c", subcore_axis_name="s")
    return pl.kernel(
        _body,
        out_type=jax.ShapeDtypeStruct((R * V,), jnp.float32),
        mesh=mesh,
        compiler_params=pltpu.CompilerParams(needs_layout_passes=False),
        scratch_types=[
            pltpu.VMEM_SHARED((N_PTS,), jnp.int32),
            pltpu.VMEM((3 * V,), jnp.float32),
            pltpu.VMEM((2, K), jnp.int32),
            pltpu.VMEM((2, K // 128, 2, 128), jnp.float32),
            pltpu.SemaphoreType.DMA,
            pltpu.SemaphoreType.DMA,
            pltpu.SemaphoreType.DMA,
            pltpu.SemaphoreType.DMA,
            pltpu.SemaphoreType.DMA,
            pltpu.SemaphoreType.DMA,
        ],
    )(seg, feat_flat)


def kernel(points, features):
    seg = _segments(points).reshape(-1)
    feat_view = (features.reshape(B, C // 8, 8, N_PTS // 128, 128)
                 .transpose(0, 1, 3, 2, 4)
                 .reshape(B * (C // 8) * (N_PTS // 128), 8, 128))
    out = _to_cminor(_voxelize(seg, feat_view))
    return (out.reshape(B, GRID, GRID, GRID, C)
            .transpose(0, 4, 1, 2, 3))


# speedup vs baseline: 1.5745x; 1.3462x over previous
import functools

import jax
import jax.numpy as jnp
import numpy as np
from jax import lax
from jax.experimental import pallas as pl
from jax.experimental.pallas import tpu as pltpu
from jax.experimental.pallas import tpu_sc as plsc

GRID = 32
V = GRID ** 3
N_PTS = 262144
B, C = 2, 64
R = B * C
NC, NS = 2, 16
NW = NC * NS
ROWS_PER_W = R // NW
L = 16

K = 2048
U = 8
VS_EPS = np.float32(np.float32(0.1) + np.float32(1e-6))

_SEG_BLK = 4096


def _seg_body(pts_ref, seg_ref):
    xi = (pts_ref[0, :] / VS_EPS).astype(jnp.int32)
    yi = (pts_ref[1, :] / VS_EPS).astype(jnp.int32)
    zi = (pts_ref[2, :] / VS_EPS).astype(jnp.int32)
    seg = xi * (GRID * GRID) + yi * GRID + zi
    seg_ref[...] = seg.reshape(_SEG_BLK // 128, 128)


@jax.jit
def _segments(pts_t):
    return pl.pallas_call(
        _seg_body,
        grid=(N_PTS // _SEG_BLK,),
        in_specs=[pl.BlockSpec((3, _SEG_BLK), lambda i: (0, i))],
        out_specs=pl.BlockSpec((_SEG_BLK // 128, 128), lambda i: (i, 0)),
        out_shape=jax.ShapeDtypeStruct((N_PTS // 128, 128), jnp.int32),
    )(pts_t)


def _tr_body(acc_ref, out_ref):
    out_ref[0, :, :] = acc_ref[...].T


@jax.jit
def _to_cminor(flat):
    return pl.pallas_call(
        _tr_body,
        grid=(B, 8),
        in_specs=[pl.BlockSpec((C, V // 8), lambda b, j: (b, j))],
        out_specs=pl.BlockSpec((1, V // 8, C), lambda b, j: (b, j, 0)),
        out_shape=jax.ShapeDtypeStruct((B, V, C), jnp.float32),
    )(flat.reshape(R, V))


def _body(seg_hbm, feat_hbm, out_hbm, seg_sh, acc, seg_buf, fbuf,
          sseg0, sseg1, sfa0, sfa1, sfb0, sfb1):
    c = lax.axis_index("c")
    s = lax.axis_index("s")
    wid = c * NS + s
    tile = wid // 2
    half = wid % 2

    ones = jnp.ones((L,), jnp.float32)
    zeros = jnp.zeros((L,), jnp.float32)

    seg_per_sub = N_PTS // NS
    pltpu.sync_copy(seg_hbm.at[pl.ds(s * seg_per_sub, seg_per_sub)],
                    seg_sh.at[pl.ds(s * seg_per_sub, seg_per_sub)])
    plsc.subcore_barrier()

    nch = N_PTS // K
    sems = ((sseg0, sfa0, sfb0), (sseg1, sfa1, sfb1))

    for p in range(2):
        row0 = 8 * tile + 4 * half + 2 * p
        nzero = 3 * V if p == 0 else 2 * V

        def zbody(i, _):
            for u in range(U):
                acc[pl.ds((i * U + u) * L, L)] = zeros
            return 0

        lax.fori_loop(0, nzero // (L * U), zbody, 0)

        def issue(j, b):
            ss, sa, sb = sems[b]
            pltpu.async_copy(seg_sh.at[pl.ds(j * K, K)], seg_buf.at[b], ss)
            pltpu.async_copy(
                feat_hbm.at[pl.ds(tile * 2048 + j * (K // 128), K // 128),
                            pl.ds(4 * half + 2 * p, 2), :],
                fbuf.at[b], sa)

        def wait(b):
            ss, sa, sb = sems[b]
            pltpu.make_async_copy(seg_sh.at[pl.ds(0, K)], seg_buf.at[b],
                                  ss).wait()
            pltpu.make_async_copy(feat_hbm.at[pl.ds(0, K // 128),
                                              pl.ds(0, 2), :],
                                  fbuf.at[b], sa).wait()

        def compute(b, with_counts):
            def ibody(i, _):
                for u in range(U):
                    seg = seg_buf[b, pl.ds(i * 128 + u * L, L)]
                    va = fbuf[b, i, 0, pl.ds(u * L, L)]
                    vb = fbuf[b, i, 1, pl.ds(u * L, L)]
                    plsc.addupdate_scatter(acc, [seg], va)
                    plsc.addupdate_scatter(acc, [seg + V], vb)
                    if with_counts:
                        plsc.addupdate_scatter(acc, [seg + 2 * V], ones)
                return 0

            lax.fori_loop(0, K // 128, ibody, 0)

        issue(0, 0)

        def mchunk(jj, _):
            for b in range(2):
                j = jj * 2 + b
                nxt = j + 1
                nxt = jnp.where(nxt >= nch, 0, nxt)
                issue(nxt, b ^ 1)
                wait(b)
                compute(b, p == 0)
            return 0

        lax.fori_loop(0, nch // 2, mchunk, 0)
        wait(0)

        def fbody(i, _):
            for u in range(4):
                bb = (i * 4 + u) * L
                cnt = jnp.maximum(acc[pl.ds(2 * V + bb, L)], 1.0)
                acc[pl.ds(bb, L)] = acc[pl.ds(bb, L)] / cnt
                acc[pl.ds(V + bb, L)] = acc[pl.ds(V + bb, L)] / cnt
            return 0

        lax.fori_loop(0, V // (L * 4), fbody, 0)
        pltpu.sync_copy(acc.at[pl.ds(0, V)], out_hbm.at[pl.ds(row0 * V, V)])
        pltpu.sync_copy(acc.at[pl.ds(V, V)],
                        out_hbm.at[pl.ds((row0 + 1) * V, V)])


@jax.jit
def _voxelize(seg, feat_flat):
    mesh = plsc.VectorSubcoreMesh(core_axis_name="c", subcore_axis_name="s")
    return pl.kernel(
        _body,
        out_type=jax.ShapeDtypeStruct((R * V,), jnp.float32),
        mesh=mesh,
        compiler_params=pltpu.CompilerParams(needs_layout_passes=False),
        scratch_types=[
            pltpu.VMEM_SHARED((N_PTS,), jnp.int32),
            pltpu.VMEM((3 * V,), jnp.float32),
            pltpu.VMEM((2, K), jnp.int32),
            pltpu.VMEM((2, K // 128, 2, 128), jnp.float32),
            pltpu.SemaphoreType.DMA,
            pltpu.SemaphoreType.DMA,
            pltpu.SemaphoreType.DMA,
            pltpu.SemaphoreType.DMA,
            pltpu.SemaphoreType.DMA,
            pltpu.SemaphoreType.DMA,
        ],
    )(seg, feat_flat)


def kernel(points, features):
    seg = _segments(points.T).reshape(-1)
    feat_view = (features.reshape(B, C // 8, 8, N_PTS // 128, 128)
                 .transpose(0, 1, 3, 2, 4)
                 .reshape(B * (C // 8) * (N_PTS // 128), 8, 128))
    out = _to_cminor(_voxelize(seg, feat_view))
    return (out.reshape(B, GRID, GRID, GRID, C)
            .transpose(0, 4, 1, 2, 3))


# speedup vs baseline: 1.5870x; 1.0079x over previous
import functools

import jax
import jax.numpy as jnp
import numpy as np
from jax import lax
from jax.experimental import pallas as pl
from jax.experimental.pallas import tpu as pltpu
from jax.experimental.pallas import tpu_sc as plsc

GRID = 32
V = GRID ** 3
N_PTS = 262144
B, C = 2, 64
R = B * C
NC, NS = 2, 16
NW = NC * NS
ROWS_PER_W = R // NW
L = 16

K = 2048
U = 8
VS_EPS = np.float32(np.float32(0.1) + np.float32(1e-6))

_SEG_BLK = 4096


def _seg_body(pts_ref, seg_ref):
    xi = (pts_ref[0, :] / VS_EPS).astype(jnp.int32)
    yi = (pts_ref[1, :] / VS_EPS).astype(jnp.int32)
    zi = (pts_ref[2, :] / VS_EPS).astype(jnp.int32)
    seg = xi * (GRID * GRID) + yi * GRID + zi
    seg_ref[...] = seg.reshape(_SEG_BLK // 128, 128)


@jax.jit
def _segments(pts_t):
    return pl.pallas_call(
        _seg_body,
        grid=(N_PTS // _SEG_BLK,),
        in_specs=[pl.BlockSpec((3, _SEG_BLK), lambda i: (0, i))],
        out_specs=pl.BlockSpec((_SEG_BLK // 128, 128), lambda i: (i, 0)),
        out_shape=jax.ShapeDtypeStruct((N_PTS // 128, 128), jnp.int32),
    )(pts_t)


def _tr_body(acc_ref, out_ref):
    out_ref[0, :, :] = acc_ref[...].T


@jax.jit
def _to_cminor(flat):
    return pl.pallas_call(
        _tr_body,
        grid=(B, 8),
        in_specs=[pl.BlockSpec((C, V // 8), lambda b, j: (b, j))],
        out_specs=pl.BlockSpec((1, V // 8, C), lambda b, j: (b, j, 0)),
        out_shape=jax.ShapeDtypeStruct((B, V, C), jnp.float32),
    )(flat.reshape(R, V))


NBUF = 4


def _body(seg_hbm, feat_hbm, out_hbm, seg_sh, cnt_sh, acc, seg0_buf, rc_buf,
          seg_buf, fbuf, sems_seg, sems_f):
    c = lax.axis_index("c")
    s = lax.axis_index("s")
    wid = c * NS + s
    tile = wid // 2
    half = wid % 2

    zeros = jnp.zeros((L,), jnp.float32)

    def zc(i, _):
        rc_buf[pl.ds(i * L, L)] = zeros
        return 0

    lax.fori_loop(0, 2048 // L, zc, 0)
    pltpu.sync_copy(rc_buf, cnt_sh.at[pl.ds(s * 2048, 2048)])
    plsc.subcore_barrier()

    def ones_i(i, _):
        rc_buf[pl.ds(i * L, L)] = zeros + 1.0
        return 0

    lax.fori_loop(0, 2048 // L, ones_i, 0)

    seg_per_sub = N_PTS // NS
    base = s * seg_per_sub
    pltpu.sync_copy(seg_hbm.at[pl.ds(base, seg_per_sub)],
                    seg_sh.at[pl.ds(base, seg_per_sub)])
    for h in range(8):
        pltpu.sync_copy(seg_hbm.at[pl.ds(base + h * 2048, 2048)], seg0_buf)
        pltpu.sync_copy(rc_buf, cnt_sh.at[seg0_buf], add=True)
    plsc.subcore_barrier()

    def rbody(i, _):
        v = rc_buf[pl.ds(i * L, L)]
        rc_buf[pl.ds(i * L, L)] = 1.0 / jnp.maximum(v, 1.0)
        return 0

    pltpu.sync_copy(cnt_sh.at[pl.ds(s * 2048, 2048)], rc_buf)
    lax.fori_loop(0, 2048 // L, rbody, 0)
    pltpu.sync_copy(rc_buf, cnt_sh.at[pl.ds(s * 2048, 2048)])
    plsc.subcore_barrier()

    nch = N_PTS // K

    for p in range(2):
        row0 = 8 * tile + 4 * half + 2 * p

        def zbody(i, _):
            for u in range(U):
                acc[pl.ds((i * U + u) * L, L)] = zeros
            return 0

        lax.fori_loop(0, 2 * V // (L * U), zbody, 0)

        def issue(j, b):
            pltpu.async_copy(seg_sh.at[pl.ds(j * K, K)], seg_buf.at[b],
                             sems_seg.at[b])
            pltpu.async_copy(
                feat_hbm.at[pl.ds(tile * 2048 + j * (K // 128), K // 128),
                            pl.ds(4 * half + 2 * p, 2), :],
                fbuf.at[b], sems_f.at[b])

        def wait(b):
            pltpu.make_async_copy(seg_sh.at[pl.ds(0, K)], seg_buf.at[b],
                                  sems_seg.at[b]).wait()
            pltpu.make_async_copy(feat_hbm.at[pl.ds(0, K // 128),
                                              pl.ds(0, 2), :],
                                  fbuf.at[b], sems_f.at[b]).wait()

        def compute(b):
            def ibody(i, _):
                for u in range(U):
                    seg = seg_buf[b, pl.ds(i * 128 + u * L, L)]
                    va = fbuf[b, i, 0, pl.ds(u * L, L)]
                    vb = fbuf[b, i, 1, pl.ds(u * L, L)]
                    plsc.addupdate_scatter(acc, [seg], va)
                    plsc.addupdate_scatter(acc, [seg + V], vb)
                return 0

            lax.fori_loop(0, K // 128, ibody, 0)

        for b in range(NBUF - 1):
            issue(b, b)

        def mchunk(jj, _):
            for b in range(NBUF):
                j = jj * NBUF + b
                nxt = j + (NBUF - 1)
                nxt = jnp.where(nxt >= nch, nxt - nch, nxt)
                issue(nxt, (b + NBUF - 1) % NBUF)
                wait(b)
                compute(b)
            return 0

        lax.fori_loop(0, nch // NBUF, mchunk, 0)
        for b in range(NBUF - 1):
            wait(b)

        def fchunk(t, _):
            pltpu.sync_copy(cnt_sh.at[pl.ds(t * 2048, 2048)], rc_buf)

            def fbody(i, _):
                bb = i * L
                rc = rc_buf[pl.ds(bb, L)]
                gbb = t * 2048 + bb
                acc[pl.ds(gbb, L)] = acc[pl.ds(gbb, L)] * rc
                acc[pl.ds(V + gbb, L)] = acc[pl.ds(V + gbb, L)] * rc
                return 0

            lax.fori_loop(0, 2048 // L, fbody, 0)
            return 0

        lax.fori_loop(0, V // 2048, fchunk, 0)
        pltpu.sync_copy(acc.at[pl.ds(0, V)], out_hbm.at[pl.ds(row0 * V, V)])
        pltpu.sync_copy(acc.at[pl.ds(V, V)],
                        out_hbm.at[pl.ds((row0 + 1) * V, V)])


@jax.jit
def _voxelize(seg, feat_flat):
    mesh = plsc.VectorSubcoreMesh(core_axis_name="c", subcore_axis_name="s")
    return pl.kernel(
        _body,
        out_type=jax.ShapeDtypeStruct((R * V,), jnp.float32),
        mesh=mesh,
        compiler_params=pltpu.CompilerParams(needs_layout_passes=False),
        scratch_types=[
            pltpu.VMEM_SHARED((N_PTS,), jnp.int32),
            pltpu.VMEM_SHARED((V,), jnp.float32),
            pltpu.VMEM((2 * V,), jnp.float32),
            pltpu.VMEM((2048,), jnp.int32),
            pltpu.VMEM((2048,), jnp.float32),
            pltpu.VMEM((NBUF, K), jnp.int32),
            pltpu.VMEM((NBUF, K // 128, 2, 128), jnp.float32),
            pltpu.SemaphoreType.DMA((NBUF,)),
            pltpu.SemaphoreType.DMA((NBUF,)),
        ],
    )(seg, feat_flat)


def kernel(points, features):
    seg = _segments(points.T).reshape(-1)
    feat_view = (features.reshape(B, C // 8, 8, N_PTS // 128, 128)
                 .transpose(0, 1, 3, 2, 4)
                 .reshape(B * (C // 8) * (N_PTS // 128), 8, 128))
    out = _to_cminor(_voxelize(seg, feat_view))
    return (out.reshape(B, GRID, GRID, GRID, C)
            .transpose(0, 4, 1, 2, 3))


# speedup vs baseline: 1.6815x; 1.0595x over previous
import functools

import jax
import jax.numpy as jnp
import numpy as np
from jax import lax
from jax.experimental import pallas as pl
from jax.experimental.pallas import tpu as pltpu
from jax.experimental.pallas import tpu_sc as plsc

GRID = 32
V = GRID ** 3
N_PTS = 262144
B, C = 2, 64
R = B * C
NC, NS = 2, 16
NW = NC * NS
ROWS_PER_W = R // NW
L = 16

K = 2048
U = 8
VS_EPS = np.float32(np.float32(0.1) + np.float32(1e-6))

_SEG_BLK = 16384


def _seg_body(pts_ref, seg_ref):
    xi = (pts_ref[0, :] / VS_EPS).astype(jnp.int32)
    yi = (pts_ref[1, :] / VS_EPS).astype(jnp.int32)
    zi = (pts_ref[2, :] / VS_EPS).astype(jnp.int32)
    seg_ref[...] = xi * (GRID * GRID) + yi * GRID + zi


@jax.jit
def _segments(pts_t):
    return pl.pallas_call(
        _seg_body,
        grid=(N_PTS // _SEG_BLK,),
        in_specs=[pl.BlockSpec((3, _SEG_BLK), lambda i: (0, i))],
        out_specs=pl.BlockSpec((_SEG_BLK,), lambda i: (i,)),
        out_shape=jax.ShapeDtypeStruct((N_PTS,), jnp.int32),
    )(pts_t)


def _tr_body(acc_ref, out_ref):
    out_ref[0, :, :] = acc_ref[...].T


@jax.jit
def _to_cminor(flat):
    return pl.pallas_call(
        _tr_body,
        grid=(B, 8),
        in_specs=[pl.BlockSpec((C, V // 8), lambda b, j: (b, j))],
        out_specs=pl.BlockSpec((1, V // 8, C), lambda b, j: (b, j, 0)),
        out_shape=jax.ShapeDtypeStruct((B, V, C), jnp.float32),
    )(flat.reshape(R, V))


NBUF = 4


def _body(seg_hbm, feat_hbm, out_hbm, seg_sh, cnt_sh, acc, seg0_buf, rc_buf,
          seg_buf, fbuf, sems_seg, sems_f):
    c = lax.axis_index("c")
    s = lax.axis_index("s")
    wid = c * NS + s
    tile = wid // 2
    half = wid % 2

    zeros = jnp.zeros((L,), jnp.float32)

    def zc(i, _):
        rc_buf[pl.ds(i * L, L)] = zeros
        return 0

    lax.fori_loop(0, 2048 // L, zc, 0)
    pltpu.sync_copy(rc_buf, cnt_sh.at[pl.ds(s * 2048, 2048)])
    plsc.subcore_barrier()

    def ones_i(i, _):
        rc_buf[pl.ds(i * L, L)] = zeros + 1.0
        return 0

    lax.fori_loop(0, 2048 // L, ones_i, 0)

    seg_per_sub = N_PTS // NS
    base = s * seg_per_sub
    pltpu.sync_copy(seg_hbm.at[pl.ds(base, seg_per_sub)],
                    seg_sh.at[pl.ds(base, seg_per_sub)])
    for h in range(8):
        pltpu.sync_copy(seg_hbm.at[pl.ds(base + h * 2048, 2048)], seg0_buf)
        pltpu.sync_copy(rc_buf, cnt_sh.at[seg0_buf], add=True)
    plsc.subcore_barrier()

    def rbody(i, _):
        v = rc_buf[pl.ds(i * L, L)]
        rc_buf[pl.ds(i * L, L)] = 1.0 / jnp.maximum(v, 1.0)
        return 0

    pltpu.sync_copy(cnt_sh.at[pl.ds(s * 2048, 2048)], rc_buf)
    lax.fori_loop(0, 2048 // L, rbody, 0)
    pltpu.sync_copy(rc_buf, cnt_sh.at[pl.ds(s * 2048, 2048)])
    plsc.subcore_barrier()

    nch = N_PTS // K

    for p in range(2):
        row0 = 8 * tile + 4 * half + 2 * p

        def zbody(i, _):
            for u in range(U):
                acc[pl.ds((i * U + u) * L, L)] = zeros
            return 0

        lax.fori_loop(0, 2 * V // (L * U), zbody, 0)

        def issue(j, b):
            pltpu.async_copy(seg_sh.at[pl.ds(j * K, K)], seg_buf.at[b],
                             sems_seg.at[b])
            pltpu.async_copy(
                feat_hbm.at[pl.ds(tile * 2048 + j * (K // 128), K // 128),
                            pl.ds(4 * half + 2 * p, 2), :],
                fbuf.at[b], sems_f.at[b])

        def wait(b):
            pltpu.make_async_copy(seg_sh.at[pl.ds(0, K)], seg_buf.at[b],
                                  sems_seg.at[b]).wait()
            pltpu.make_async_copy(feat_hbm.at[pl.ds(0, K // 128),
                                              pl.ds(0, 2), :],
                                  fbuf.at[b], sems_f.at[b]).wait()

        def compute(b):
            def ibody(i, _):
                for u in range(U):
                    seg = seg_buf[b, pl.ds(i * 128 + u * L, L)]
                    va = fbuf[b, i, 0, pl.ds(u * L, L)]
                    vb = fbuf[b, i, 1, pl.ds(u * L, L)]
                    plsc.addupdate_scatter(acc, [seg], va)
                    plsc.addupdate_scatter(acc, [seg + V], vb)
                return 0

            lax.fori_loop(0, K // 128, ibody, 0)

        for b in range(NBUF - 1):
            issue(b, b)

        def mchunk(jj, _):
            for b in range(NBUF):
                j = jj * NBUF + b
                nxt = j + (NBUF - 1)
                nxt = jnp.where(nxt >= nch, nxt - nch, nxt)
                issue(nxt, (b + NBUF - 1) % NBUF)
                wait(b)
                compute(b)
            return 0

        lax.fori_loop(0, nch // NBUF, mchunk, 0)
        for b in range(NBUF - 1):
            wait(b)

        def fchunk(t, _):
            pltpu.sync_copy(cnt_sh.at[pl.ds(t * 2048, 2048)], rc_buf)

            def fbody(i, _):
                bb = i * L
                rc = rc_buf[pl.ds(bb, L)]
                gbb = t * 2048 + bb
                acc[pl.ds(gbb, L)] = acc[pl.ds(gbb, L)] * rc
                acc[pl.ds(V + gbb, L)] = acc[pl.ds(V + gbb, L)] * rc
                return 0

            lax.fori_loop(0, 2048 // L, fbody, 0)
            return 0

        lax.fori_loop(0, V // 2048, fchunk, 0)
        pltpu.sync_copy(acc.at[pl.ds(0, V)], out_hbm.at[pl.ds(row0 * V, V)])
        pltpu.sync_copy(acc.at[pl.ds(V, V)],
                        out_hbm.at[pl.ds((row0 + 1) * V, V)])


@jax.jit
def _voxelize(seg, feat_flat):
    mesh = plsc.VectorSubcoreMesh(core_axis_name="c", subcore_axis_name="s")
    return pl.kernel(
        _body,
        out_type=jax.ShapeDtypeStruct((R * V,), jnp.float32),
        mesh=mesh,
        compiler_params=pltpu.CompilerParams(needs_layout_passes=False),
        scratch_types=[
            pltpu.VMEM_SHARED((N_PTS,), jnp.int32),
            pltpu.VMEM_SHARED((V,), jnp.float32),
            pltpu.VMEM((2 * V,), jnp.float32),
            pltpu.VMEM((2048,), jnp.int32),
            pltpu.VMEM((2048,), jnp.float32),
            pltpu.VMEM((NBUF, K), jnp.int32),
            pltpu.VMEM((NBUF, K // 128, 2, 128), jnp.float32),
            pltpu.SemaphoreType.DMA((NBUF,)),
            pltpu.SemaphoreType.DMA((NBUF,)),
        ],
    )(seg, feat_flat)


def kernel(points, features):
    seg = _segments(points.T)
    feat_view = (features.reshape(B, C // 8, 8, N_PTS // 128, 128)
                 .transpose(0, 1, 3, 2, 4)
                 .reshape(B * (C // 8) * (N_PTS // 128), 8, 128))
    out = _to_cminor(_voxelize(seg, feat_view))
    return (out.reshape(B, GRID, GRID, GRID, C)
            .transpose(0, 4, 1, 2, 3))
